# Initial kernel scaffold; baseline (speedup 1.0000x reference)
import functools

import jax
import jax.numpy as jnp
from jax import lax
from jax.experimental import pallas as pl
from jax.experimental.pallas import tpu as pltpu
from jax.experimental.pallas import tpu_sc as plsc

N = 100000
D = 128
H = 64
G = 1024

N_PAD = 102400
AUG = 144
GA = 1040
NW = 32
K = 128
T = N_PAD // (NW * K)
RPW = T * K
BLK = 1024


def _score_kernel(x_ref, w1_ref, b1_ref, w2_ref, b2_ref, aug_ref):
    x = x_ref[...]
    w2 = w2_ref[...]
    b2 = b2_ref[0, 0]
    h = jnp.tanh(
        jnp.dot(x, w1_ref[...], preferred_element_type=jnp.float32)
        + b1_ref[...]
    )
    s = jnp.sum(h * w2, axis=1, keepdims=True) + b2
    m = jnp.sum(jnp.abs(w2)) + jnp.abs(b2)
    e = jnp.exp(s - m)
    lane16 = lax.broadcasted_iota(jnp.int32, (1, 16), 1)
    tail = e * jnp.where(lane16 == 0, 1.0, 0.0)
    aug_ref[...] = jnp.concatenate([x * e, tail], axis=1)


def _final_kernel(p_ref, out_ref):
    a = p_ref[0] + p_ref[1]
    num = a[0:G, 0:D]
    den = a[0:G, D:D + 1]
    out_ref[...] = jnp.where(den > 0.0, num / den, 0.0)


def _make_scatter_kernel():
    mesh = plsc.VectorSubcoreMesh(core_axis_name="c", subcore_axis_name="s")

    @functools.partial(
        pl.kernel,
        out_type=jax.ShapeDtypeStruct((2, GA, AUG), jnp.float32),
        mesh=mesh,
        scratch_types=[
            pltpu.VMEM((K, AUG), jnp.float32),
            pltpu.VMEM((T, K), jnp.int32),
            pltpu.VMEM_SHARED((GA, AUG), jnp.float32),
            pltpu.SemaphoreType.DMA,
        ],
    )
    def scatter(aug_hbm, idx_hbm, zeros_hbm, out_hbm, buf, idx_v, acc, sem):
        cid = lax.axis_index("c")
        sid = lax.axis_index("s")
        wid = cid * 16 + sid

        @pl.when(sid == 0)
        def _():
            pltpu.sync_copy(zeros_hbm, acc)

        pltpu.sync_copy(idx_hbm.at[wid], idx_v)
        plsc.subcore_barrier()

        base = wid * RPW

        @pl.loop(0, T)
        def _(t):
            pltpu.sync_copy(aug_hbm.at[pl.ds(base + t * K, K)], buf)
            pltpu.sync_copy(buf, acc.at[idx_v.at[t]], add=True)

        plsc.subcore_barrier()
        pltpu.sync_copy(
            acc.at[pl.ds(sid * (GA // 16), GA // 16)],
            out_hbm.at[cid, pl.ds(sid * (GA // 16), GA // 16)],
        )

    return scatter


_scatter = _make_scatter_kernel()


def kernel(x, batch_idx, W1, b1, W2, b2):
    xp = jnp.pad(x, ((0, N_PAD - N), (0, 0)))
    idx = jnp.full((N_PAD,), G, jnp.int32).at[:N].set(
        batch_idx.astype(jnp.int32))
    idx3 = idx.reshape(NW, T, K)
    w2t = W2.reshape(1, H)
    b1r = b1.reshape(1, H)
    b2r = b2.reshape(1, 1)

    aug = pl.pallas_call(
        _score_kernel,
        grid=(N_PAD // BLK,),
        in_specs=[
            pl.BlockSpec((BLK, D), lambda i: (i, 0)),
            pl.BlockSpec((D, H), lambda i: (0, 0)),
            pl.BlockSpec((1, H), lambda i: (0, 0)),
            pl.BlockSpec((1, H), lambda i: (0, 0)),
            pl.BlockSpec((1, 1), lambda i: (0, 0)),
        ],
        out_specs=pl.BlockSpec((BLK, AUG), lambda i: (i, 0)),
        out_shape=jax.ShapeDtypeStruct((N_PAD, AUG), jnp.float32),
    )(xp, W1, b1r, w2t, b2r)

    zeros = jnp.zeros((GA, AUG), jnp.float32)
    partials = _scatter(aug, idx3, zeros)

    pooled = pl.pallas_call(
        _final_kernel,
        grid=(1,),
        in_specs=[pl.BlockSpec((2, GA, AUG), lambda i: (0, 0, 0))],
        out_specs=pl.BlockSpec((G, D), lambda i: (0, 0)),
        out_shape=jax.ShapeDtypeStruct((G, D), jnp.float32),
    )(partials)
    return pooled


# speedup vs baseline: 8.8758x; 8.8758x over previous
import dataclasses
import functools

import jax
import jax.numpy as jnp
from jax import lax
from jax.experimental import pallas as pl
from jax.experimental.pallas import tpu as pltpu
from jax.experimental.pallas import tpu_sc as plsc

N = 100000
D = 128
H = 64
G = 1024

NW = 32
K = 128
T = 25
RPW = T * K
N_PAD = NW * RPW
GA = 1040
DEN_R = 16


def _score_kernel(x_ref, w1_ref, b1_ref, w2_ref, b2_ref, ex_ref, e_ref):
    x = x_ref[...]
    w2 = w2_ref[...]
    b2 = b2_ref[0, 0]
    m = jnp.sum(jnp.abs(w2)) + jnp.abs(b2)
    h = jnp.tanh(
        jnp.dot(x, w1_ref[...], preferred_element_type=jnp.float32)
        + b1_ref[...]
    )
    s_col = jnp.sum(h * w2, axis=1, keepdims=True) + b2
    ex_ref[...] = x * jnp.exp(s_col - m)
    s_row = lax.dot_general(
        w2, h, (((1,), (1,)), ((), ())),
        preferred_element_type=jnp.float32) + b2
    e_ref[...] = jnp.exp(s_row - m).reshape(1, 1, RPW)


def _final_kernel(p_ref, d_ref, out_ref):
    num = (p_ref[0] + p_ref[1])[0:G, :]
    den = (d_ref[0] + d_ref[1])[0:G, :]
    out_ref[...] = jnp.where(den > 0.0, num / den, 0.0)


@functools.lru_cache(maxsize=None)
def _make_scatter_kernel():
    mesh = plsc.VectorSubcoreMesh(core_axis_name="c", subcore_axis_name="s")
    cp = pltpu.CompilerParams()
    if "needs_layout_passes" in pltpu.CompilerParams.__dataclass_fields__:
        cp = dataclasses.replace(cp, needs_layout_passes=False)

    @functools.partial(
        pl.kernel,
        compiler_params=cp,
        out_type=(
            jax.ShapeDtypeStruct((2, GA, D), jnp.float32),
            jax.ShapeDtypeStruct((2, GA, D), jnp.float32),
        ),
        mesh=mesh,
        scratch_types=[
            pltpu.VMEM((K, D), jnp.float32),
            pltpu.VMEM((T, K), jnp.int32),
            pltpu.VMEM((RPW,), jnp.float32),
            pltpu.VMEM((DEN_R, D), jnp.float32),
            pltpu.VMEM((DEN_R, D), jnp.float32),
            pltpu.VMEM_SHARED((GA, D), jnp.float32),
            pltpu.VMEM_SHARED((DEN_R, D), jnp.float32),
            pltpu.SemaphoreType.DMA,
        ],
    )
    def scatter(ex_hbm, idx_hbm, e_hbm, zeros_hbm, num_out, den_out,
                buf, idx_v, e_v, den_l, den_m, acc, den_s, sem):
        cid = lax.axis_index("c")
        sid = lax.axis_index("s")
        wid = cid * 16 + sid

        @pl.when(sid == 0)
        def _():
            pltpu.sync_copy(zeros_hbm, acc)
            pltpu.sync_copy(zeros_hbm.at[pl.ds(0, DEN_R)], den_s)

        pltpu.sync_copy(idx_hbm.at[wid], idx_v)
        pltpu.sync_copy(e_hbm.at[wid], e_v)

        zero16 = jnp.zeros((16,), jnp.float32)
        for r in range(DEN_R):
            for c in range(D // 16):
                den_l[r, pl.ds(c * 16, 16)] = zero16

        plsc.subcore_barrier()

        base = wid * RPW

        @pl.loop(0, T)
        def _(t):
            pltpu.sync_copy(ex_hbm.at[pl.ds(base + t * K, K)], buf)
            pltpu.sync_copy(buf, acc.at[idx_v.at[t]], add=True)

        @pl.loop(0, RPW // 16)
        def _(i):
            e16 = e_v[pl.ds(i * 16, 16)]
            t = i // 8
            lane = (i % 8) * 16
            idx16 = idx_v[t, pl.ds(lane, 16)]
            hi = lax.shift_right_logical(idx16, 7)
            lo = lax.bitwise_and(idx16, 127)
            plsc.addupdate_scatter(den_l, [hi, lo], e16)

        rows16 = lax.iota(jnp.int32, 16)
        pltpu.sync_copy(den_l, den_s.at[rows16], add=True)
        plsc.subcore_barrier()

        pltpu.sync_copy(
            acc.at[pl.ds(sid * 64, 64)],
            num_out.at[cid, pl.ds(sid * 64, 64)],
        )

        @pl.when(sid == 0)
        def _():
            pltpu.sync_copy(
                acc.at[pl.ds(1024, GA - 1024)],
                num_out.at[cid, pl.ds(1024, GA - 1024)],
            )

        pltpu.sync_copy(den_s, den_m)

        lanes16 = lax.iota(jnp.int32, 16)

        def bcast_16rows(g0, buf_row0):
            v = den_m[g0 // 128, pl.ds((g0 % 128) // 16 * 16, 16)]
            for r in range(16):
                oh = (lanes16 == r).astype(jnp.float32)
                row = jnp.full((16,), jnp.sum(v * oh), jnp.float32)
                for c in range(D // 16):
                    buf[buf_row0 + r, pl.ds(c * 16, 16)] = row

        sbase = sid * 64

        @pl.loop(0, 4)
        def _(q):
            bcast_16rows(sbase + q * 16, q * 16)

        pltpu.sync_copy(
            buf.at[pl.ds(0, 64)],
            den_out.at[cid, pl.ds(sid * 64, 64)],
        )

        @pl.when(sid == 0)
        def _():
            bcast_16rows(1024, 0)
            pltpu.sync_copy(
                buf.at[pl.ds(0, GA - 1024)],
                den_out.at[cid, pl.ds(1024, GA - 1024)],
            )

    return scatter


def kernel(x, batch_idx, W1, b1, W2, b2):
    xp = jnp.pad(x, ((0, N_PAD - N), (0, 0)))
    idx = jnp.full((N_PAD,), G, jnp.int32).at[:N].set(
        batch_idx.astype(jnp.int32))
    idx3 = idx.reshape(NW, T, K)
    w2t = W2.reshape(1, H)
    b1r = b1.reshape(1, H)
    b2r = b2.reshape(1, 1)

    ex, e_rows = pl.pallas_call(
        _score_kernel,
        grid=(NW,),
        in_specs=[
            pl.BlockSpec((RPW, D), lambda i: (i, 0)),
            pl.BlockSpec((D, H), lambda i: (0, 0)),
            pl.BlockSpec((1, H), lambda i: (0, 0)),
            pl.BlockSpec((1, H), lambda i: (0, 0)),
            pl.BlockSpec((1, 1), lambda i: (0, 0)),
        ],
        out_specs=[
            pl.BlockSpec((RPW, D), lambda i: (i, 0)),
            pl.BlockSpec((1, 1, RPW), lambda i: (i, 0, 0)),
        ],
        out_shape=[
            jax.ShapeDtypeStruct((N_PAD, D), jnp.float32),
            jax.ShapeDtypeStruct((NW, 1, RPW), jnp.float32),
        ],
    )(xp, W1, b1r, w2t, b2r)
    e_rows = e_rows.reshape(NW, RPW)

    zeros = jnp.zeros((GA, D), jnp.float32)
    num_p, den_p = _make_scatter_kernel()(ex, idx3, e_rows, zeros)

    pooled = pl.pallas_call(
        _final_kernel,
        grid=(1,),
        in_specs=[
            pl.BlockSpec((2, GA, D), lambda i: (0, 0, 0)),
            pl.BlockSpec((2, GA, D), lambda i: (0, 0, 0)),
        ],
        out_specs=pl.BlockSpec((G, D), lambda i: (0, 0)),
        out_shape=jax.ShapeDtypeStruct((G, D), jnp.float32),
    )(num_p, den_p)
    return pooled


# speedup vs baseline: 12.1950x; 1.3740x over previous
import dataclasses
import functools

import jax
import jax.numpy as jnp
from jax import lax
from jax.experimental import pallas as pl
from jax.experimental.pallas import tpu as pltpu
from jax.experimental.pallas import tpu_sc as plsc

N = 100000
D = 128
H = 64
G = 1024

NW = 32
K = 128
T = 25
RPW = T * K
N_PAD = NW * RPW
GA = 1040
DEN_R = 16


def _score_kernel(x_ref, w1_ref, b1_ref, w2_ref, b2_ref, ex_ref, e_ref):
    x = x_ref[...]
    w2 = w2_ref[...]
    b2 = b2_ref[0, 0]
    m = jnp.sum(jnp.abs(w2)) + jnp.abs(b2)
    h = jnp.tanh(
        jnp.dot(x, w1_ref[...], preferred_element_type=jnp.float32)
        + b1_ref[...]
    )
    s_col = jnp.sum(h * w2, axis=1, keepdims=True) + b2
    ex_ref[...] = x * jnp.exp(s_col - m)
    s_row = lax.dot_general(
        w2, h, (((1,), (1,)), ((), ())),
        preferred_element_type=jnp.float32) + b2
    e_ref[...] = jnp.exp(s_row - m).reshape(1, 1, RPW)


def _final_kernel(p_ref, d_ref, out_ref):
    num = (p_ref[0] + p_ref[1])[0:G, :]
    den = (d_ref[0] + d_ref[1])[0:G, :]
    out_ref[...] = jnp.where(den > 0.0, num / den, 0.0)


@functools.lru_cache(maxsize=None)
def _make_scatter_kernel():
    mesh = plsc.VectorSubcoreMesh(core_axis_name="c", subcore_axis_name="s")
    cp = pltpu.CompilerParams()
    if "needs_layout_passes" in pltpu.CompilerParams.__dataclass_fields__:
        cp = dataclasses.replace(cp, needs_layout_passes=False)

    @functools.partial(
        pl.kernel,
        compiler_params=cp,
        out_type=(
            jax.ShapeDtypeStruct((2, GA, D), jnp.float32),
            jax.ShapeDtypeStruct((2, GA, D), jnp.float32),
        ),
        mesh=mesh,
        scratch_types=[
            pltpu.VMEM((2, K, D), jnp.float32),
            pltpu.VMEM((T, K), jnp.int32),
            pltpu.VMEM((RPW,), jnp.float32),
            pltpu.VMEM((DEN_R, D), jnp.float32),
            pltpu.VMEM((DEN_R, D), jnp.float32),
            pltpu.VMEM_SHARED((GA, D), jnp.float32),
            pltpu.VMEM_SHARED((DEN_R, D), jnp.float32),
            pltpu.SemaphoreType.DMA,
        ],
    )
    def scatter(ex_hbm, idx_hbm, e_hbm, zeros_hbm, num_out, den_out,
                buf, idx_v, e_v, den_l, den_m, acc, den_s, sem):
        cid = lax.axis_index("c")
        sid = lax.axis_index("s")
        wid = cid * 16 + sid

        @pl.when(sid == 0)
        def _():
            pltpu.sync_copy(zeros_hbm, acc)
            pltpu.sync_copy(zeros_hbm.at[pl.ds(0, DEN_R)], den_s)

        pltpu.sync_copy(idx_hbm.at[wid], idx_v)
        pltpu.sync_copy(e_hbm.at[wid], e_v)

        zero16 = jnp.zeros((16,), jnp.float32)
        for r in range(DEN_R):
            for c in range(D // 16):
                den_l[r, pl.ds(c * 16, 16)] = zero16

        plsc.subcore_barrier()

        base = wid * RPW

        def load_copy(t, b):
            return pltpu.make_async_copy(
                ex_hbm.at[pl.ds(base + t * K, K)], buf.at[b], sem)

        load_copy(0, 0).start()

        @pl.loop(0, T)
        def _(t):
            b = t % 2
            load_copy(t, b).wait()

            @pl.when(t < T - 1)
            def _():
                load_copy(t + 1, 1 - b).start()

            pltpu.sync_copy(buf.at[b], acc.at[idx_v.at[t]], add=True)

        @pl.loop(0, RPW // 16)
        def _(i):
            e16 = e_v[pl.ds(i * 16, 16)]
            t = i // 8
            lane = (i % 8) * 16
            idx16 = idx_v[t, pl.ds(lane, 16)]
            hi = lax.shift_right_logical(idx16, 7)
            lo = lax.bitwise_and(idx16, 127)
            plsc.addupdate_scatter(den_l, [hi, lo], e16)

        rows16 = lax.iota(jnp.int32, 16)
        pltpu.sync_copy(den_l, den_s.at[rows16], add=True)
        plsc.subcore_barrier()

        pltpu.sync_copy(
            acc.at[pl.ds(sid * 64, 64)],
            num_out.at[cid, pl.ds(sid * 64, 64)],
        )

        @pl.when(sid == 0)
        def _():
            pltpu.sync_copy(
                acc.at[pl.ds(1024, GA - 1024)],
                num_out.at[cid, pl.ds(1024, GA - 1024)],
            )

        pltpu.sync_copy(den_s, den_m)

        lanes16 = lax.iota(jnp.int32, 16)

        def bcast_16rows(g0, buf_row0):
            v = den_m[g0 // 128, pl.ds((g0 % 128) // 16 * 16, 16)]
            for r in range(16):
                oh = (lanes16 == r).astype(jnp.float32)
                row = jnp.full((16,), jnp.sum(v * oh), jnp.float32)
                for c in range(D // 16):
                    buf[0, buf_row0 + r, pl.ds(c * 16, 16)] = row

        sbase = sid * 64

        @pl.loop(0, 4)
        def _(q):
            bcast_16rows(sbase + q * 16, q * 16)

        pltpu.sync_copy(
            buf.at[0, pl.ds(0, 64)],
            den_out.at[cid, pl.ds(sid * 64, 64)],
        )

        @pl.when(sid == 0)
        def _():
            bcast_16rows(1024, 0)
            pltpu.sync_copy(
                buf.at[0, pl.ds(0, GA - 1024)],
                den_out.at[cid, pl.ds(1024, GA - 1024)],
            )

    return scatter


def kernel(x, batch_idx, W1, b1, W2, b2):
    idx = jnp.full((N_PAD,), G, jnp.int32).at[:N].set(
        batch_idx.astype(jnp.int32))
    idx3 = idx.reshape(NW, T, K)
    w2t = W2.reshape(1, H)
    b1r = b1.reshape(1, H)
    b2r = b2.reshape(1, 1)

    ex, e_rows = pl.pallas_call(
        _score_kernel,
        grid=(NW,),
        in_specs=[
            pl.BlockSpec((RPW, D), lambda i: (i, 0)),
            pl.BlockSpec((D, H), lambda i: (0, 0)),
            pl.BlockSpec((1, H), lambda i: (0, 0)),
            pl.BlockSpec((1, H), lambda i: (0, 0)),
            pl.BlockSpec((1, 1), lambda i: (0, 0)),
        ],
        out_specs=[
            pl.BlockSpec((RPW, D), lambda i: (i, 0)),
            pl.BlockSpec((1, 1, RPW), lambda i: (i, 0, 0)),
        ],
        out_shape=[
            jax.ShapeDtypeStruct((N_PAD, D), jnp.float32),
            jax.ShapeDtypeStruct((NW, 1, RPW), jnp.float32),
        ],
    )(x, W1, b1r, w2t, b2r)
    e_rows = e_rows.reshape(NW, RPW)

    zeros = jnp.zeros((GA, D), jnp.float32)
    num_p, den_p = _make_scatter_kernel()(ex, idx3, e_rows, zeros)

    pooled = pl.pallas_call(
        _final_kernel,
        grid=(1,),
        in_specs=[
            pl.BlockSpec((2, GA, D), lambda i: (0, 0, 0)),
            pl.BlockSpec((2, GA, D), lambda i: (0, 0, 0)),
        ],
        out_specs=pl.BlockSpec((G, D), lambda i: (0, 0)),
        out_shape=jax.ShapeDtypeStruct((G, D), jnp.float32),
    )(num_p, den_p)
    return pooled


# speedup vs baseline: 12.2168x; 1.0018x over previous
import dataclasses
import functools

import jax
import jax.numpy as jnp
from jax import lax
from jax.experimental import pallas as pl
from jax.experimental.pallas import tpu as pltpu
from jax.experimental.pallas import tpu_sc as plsc

N = 100000
D = 128
H = 64
G = 1024

NW = 32
K = 128
T = 25
RPW = T * K
N_PAD = NW * RPW
GA = 1040
DEN_R = 16


def _score_kernel(x_ref, w1_ref, b1_ref, w2_ref, b2_ref, ex_ref, e_ref):
    x = x_ref[...]
    w2 = w2_ref[...]
    b2 = b2_ref[0, 0]
    m = jnp.sum(jnp.abs(w2)) + jnp.abs(b2)
    h = jnp.tanh(
        jnp.dot(x, w1_ref[...], preferred_element_type=jnp.float32)
        + b1_ref[...]
    )
    s_col = jnp.sum(h * w2, axis=1, keepdims=True) + b2
    ex_ref[...] = x * jnp.exp(s_col - m)
    s_row = lax.dot_general(
        w2, h, (((1,), (1,)), ((), ())),
        preferred_element_type=jnp.float32) + b2
    e_ref[...] = jnp.exp(s_row - m).reshape(1, 1, RPW)


def _final_kernel(p_ref, d_ref, out_ref):
    num = (p_ref[0] + p_ref[1])[0:G, :]
    den = (d_ref[0] + d_ref[1])[0:G, :]
    out_ref[...] = jnp.where(den > 0.0, num / den, 0.0)


@functools.lru_cache(maxsize=None)
def _make_scatter_kernel():
    mesh = plsc.VectorSubcoreMesh(core_axis_name="c", subcore_axis_name="s")
    cp = pltpu.CompilerParams()
    if "needs_layout_passes" in pltpu.CompilerParams.__dataclass_fields__:
        cp = dataclasses.replace(cp, needs_layout_passes=False)

    @functools.partial(
        pl.kernel,
        compiler_params=cp,
        out_type=(
            jax.ShapeDtypeStruct((2, GA, D), jnp.float32),
            jax.ShapeDtypeStruct((2, GA, D), jnp.float32),
        ),
        mesh=mesh,
        scratch_types=[
            pltpu.VMEM((2, K, D), jnp.float32),
            pltpu.VMEM((T, K), jnp.int32),
            pltpu.VMEM((RPW,), jnp.float32),
            pltpu.VMEM((DEN_R, D), jnp.float32),
            pltpu.VMEM((DEN_R, D), jnp.float32),
            pltpu.VMEM_SHARED((GA, D), jnp.float32),
            pltpu.VMEM_SHARED((DEN_R, D), jnp.float32),
            pltpu.SemaphoreType.DMA,
        ],
    )
    def scatter(ex_hbm, idx_hbm, e_hbm, zeros_hbm, num_out, den_out,
                buf, idx_v, e_v, den_l, den_m, acc, den_s, sem):
        cid = lax.axis_index("c")
        sid = lax.axis_index("s")
        wid = cid * 16 + sid

        @pl.when(sid == 0)
        def _():
            pltpu.sync_copy(zeros_hbm, acc)
            pltpu.sync_copy(zeros_hbm.at[pl.ds(0, DEN_R)], den_s)

        pltpu.sync_copy(idx_hbm.at[wid], idx_v)
        pltpu.sync_copy(e_hbm.at[wid], e_v)

        zero16 = jnp.zeros((16,), jnp.float32)
        for r in range(DEN_R):
            for c in range(D // 16):
                den_l[r, pl.ds(c * 16, 16)] = zero16

        plsc.subcore_barrier()

        base = wid * RPW

        def load_copy(t, b):
            return pltpu.make_async_copy(
                ex_hbm.at[pl.ds(base + t * K, K)], buf.at[b], sem)

        load_copy(0, 0).start()

        @pl.loop(0, T)
        def _(t):
            b = t % 2
            load_copy(t, b).wait()

            @pl.when(t < T - 1)
            def _():
                load_copy(t + 1, 1 - b).start()

            pltpu.sync_copy(buf.at[b], acc.at[idx_v.at[t]], add=True)

        @pl.loop(0, RPW // 16)
        def _(i):
            e16 = e_v[pl.ds(i * 16, 16)]
            t = i // 8
            lane = (i % 8) * 16
            idx16 = idx_v[t, pl.ds(lane, 16)]
            hi = lax.shift_right_logical(idx16, 7)
            lo = lax.bitwise_and(idx16, 127)
            plsc.addupdate_scatter(den_l, [hi, lo], e16)

        rows16 = lax.iota(jnp.int32, 16)
        pltpu.sync_copy(den_l, den_s.at[rows16], add=True)
        plsc.subcore_barrier()

        pltpu.sync_copy(
            acc.at[pl.ds(sid * 64, 64)],
            num_out.at[cid, pl.ds(sid * 64, 64)],
        )

        @pl.when(sid == 0)
        def _():
            pltpu.sync_copy(
                acc.at[pl.ds(1024, GA - 1024)],
                num_out.at[cid, pl.ds(1024, GA - 1024)],
            )

        pltpu.sync_copy(den_s, den_m)

        lanes16 = lax.iota(jnp.int32, 16)

        def bcast_16rows(g0, buf_row0):
            v = den_m[g0 // 128, pl.ds((g0 % 128) // 16 * 16, 16)]
            for r in range(16):
                oh = (lanes16 == r).astype(jnp.float32)
                row = jnp.full((16,), jnp.sum(v * oh), jnp.float32)
                for c in range(D // 16):
                    buf[0, buf_row0 + r, pl.ds(c * 16, 16)] = row

        sbase = sid * 64

        @pl.loop(0, 4)
        def _(q):
            bcast_16rows(sbase + q * 16, q * 16)

        pltpu.sync_copy(
            buf.at[0, pl.ds(0, 64)],
            den_out.at[cid, pl.ds(sid * 64, 64)],
        )

        @pl.when(sid == 0)
        def _():
            bcast_16rows(1024, 0)
            pltpu.sync_copy(
                buf.at[0, pl.ds(0, GA - 1024)],
                den_out.at[cid, pl.ds(1024, GA - 1024)],
            )

    return scatter


def kernel(x, batch_idx, W1, b1, W2, b2):
    idx3 = jnp.pad(batch_idx.astype(jnp.int32), (0, N_PAD - N),
                   constant_values=G).reshape(NW, T, K)
    w2t = W2.reshape(1, H)
    b1r = b1.reshape(1, H)
    b2r = b2.reshape(1, 1)

    ex, e_rows = pl.pallas_call(
        _score_kernel,
        grid=(NW,),
        in_specs=[
            pl.BlockSpec((RPW, D), lambda i: (i, 0)),
            pl.BlockSpec((D, H), lambda i: (0, 0)),
            pl.BlockSpec((1, H), lambda i: (0, 0)),
            pl.BlockSpec((1, H), lambda i: (0, 0)),
            pl.BlockSpec((1, 1), lambda i: (0, 0)),
        ],
        out_specs=[
            pl.BlockSpec((RPW, D), lambda i: (i, 0)),
            pl.BlockSpec((1, 1, RPW), lambda i: (i, 0, 0)),
        ],
        out_shape=[
            jax.ShapeDtypeStruct((N_PAD, D), jnp.float32),
            jax.ShapeDtypeStruct((NW, 1, RPW), jnp.float32),
        ],
        compiler_params=pltpu.CompilerParams(
            dimension_semantics=("parallel",)),
    )(x, W1, b1r, w2t, b2r)
    e_rows = e_rows.reshape(NW, RPW)

    zeros = jnp.zeros((GA, D), jnp.float32)
    num_p, den_p = _make_scatter_kernel()(ex, idx3, e_rows, zeros)

    pooled = pl.pallas_call(
        _final_kernel,
        grid=(1,),
        in_specs=[
            pl.BlockSpec((2, GA, D), lambda i: (0, 0, 0)),
            pl.BlockSpec((2, GA, D), lambda i: (0, 0, 0)),
        ],
        out_specs=pl.BlockSpec((G, D), lambda i: (0, 0)),
        out_shape=jax.ShapeDtypeStruct((G, D), jnp.float32),
    )(num_p, den_p)
    return pooled


# speedup vs baseline: 12.7373x; 1.0426x over previous
import dataclasses
import functools

import jax
import jax.numpy as jnp
from jax import lax
from jax.experimental import pallas as pl
from jax.experimental.pallas import tpu as pltpu
from jax.experimental.pallas import tpu_sc as plsc

N = 100000
D = 128
H = 64
G = 1024

NW = 32
K = 64
T = 50
NBUF = 8
RPW = T * K
N_PAD = NW * RPW
GA = 1040
DEN_R = 16


def _score_kernel(x_ref, w1_ref, b1_ref, w2_ref, b2_ref, ex_ref, e_ref):
    x = x_ref[...]
    w2 = w2_ref[...]
    b2 = b2_ref[0, 0]
    m = jnp.sum(jnp.abs(w2)) + jnp.abs(b2)
    h = jnp.tanh(
        jnp.dot(x, w1_ref[...], preferred_element_type=jnp.float32)
        + b1_ref[...]
    )
    s_col = jnp.sum(h * w2, axis=1, keepdims=True) + b2
    ex_ref[...] = x * jnp.exp(s_col - m)
    s_row = lax.dot_general(
        w2, h, (((1,), (1,)), ((), ())),
        preferred_element_type=jnp.float32) + b2
    e_ref[...] = jnp.exp(s_row - m).reshape(1, 1, RPW)


def _final_kernel(p_ref, d_ref, out_ref):
    num = (p_ref[0] + p_ref[1])[0:G, :]
    den = (d_ref[0] + d_ref[1])[0:G, :]
    out_ref[...] = jnp.where(den > 0.0, num / den, 0.0)


@functools.lru_cache(maxsize=None)
def _make_scatter_kernel():
    mesh = plsc.VectorSubcoreMesh(core_axis_name="c", subcore_axis_name="s")
    cp = pltpu.CompilerParams()
    if "needs_layout_passes" in pltpu.CompilerParams.__dataclass_fields__:
        cp = dataclasses.replace(cp, needs_layout_passes=False)

    @functools.partial(
        pl.kernel,
        compiler_params=cp,
        out_type=(
            jax.ShapeDtypeStruct((2, GA, D), jnp.float32),
            jax.ShapeDtypeStruct((2, GA, D), jnp.float32),
        ),
        mesh=mesh,
        scratch_types=[
            pltpu.VMEM((NBUF, K, D), jnp.float32),
            pltpu.VMEM((T, K), jnp.int32),
            pltpu.VMEM((RPW,), jnp.float32),
            pltpu.VMEM((DEN_R, D), jnp.float32),
            pltpu.VMEM((DEN_R, D), jnp.float32),
            pltpu.VMEM_SHARED((GA, D), jnp.float32),
            pltpu.VMEM_SHARED((DEN_R, D), jnp.float32),
            pltpu.SemaphoreType.DMA,
            pltpu.SemaphoreType.DMA((4,)),
            pltpu.SemaphoreType.DMA((4,)),
        ],
    )
    def scatter(ex_hbm, idx_hbm, e_hbm, zeros_hbm, num_out, den_out,
                buf, idx_v, e_v, den_l, den_m, acc, den_s, sem, lsem, ssem):
        cid = lax.axis_index("c")
        sid = lax.axis_index("s")
        wid = cid * 16 + sid

        @pl.when(sid == 0)
        def _():
            pltpu.sync_copy(zeros_hbm, acc)
            pltpu.sync_copy(zeros_hbm.at[pl.ds(0, DEN_R)], den_s)

        pltpu.sync_copy(idx_hbm.at[wid], idx_v)
        pltpu.sync_copy(e_hbm.at[wid], e_v)

        zero16 = jnp.zeros((16,), jnp.float32)
        for r in range(DEN_R):
            for c in range(D // 16):
                den_l[r, pl.ds(c * 16, 16)] = zero16

        plsc.subcore_barrier()

        base = wid * RPW

        def load_copy(t):
            return pltpu.make_async_copy(
                ex_hbm.at[pl.ds(base + t * K, K)], buf.at[t % NBUF],
                lsem.at[t % 4])

        def scat_copy(t):
            return pltpu.make_async_copy(
                buf.at[t % NBUF], acc.at[idx_v.at[t]], ssem.at[t % 4])

        for t in range(4):
            load_copy(t).start()

        @pl.loop(0, T // 2)
        def _(p):
            for t in (2 * p, 2 * p + 1):
                load_copy(t).wait()

                @pl.when(p >= 2)
                def _():
                    scat_copy(t - 4).wait()

                scat_copy(t).start(add=True)

                @pl.when(p < T // 2 - 2)
                def _():
                    load_copy(t + 4).start()

        for q in range(4):
            scat_copy(T - 4 + q).wait()

        @pl.loop(0, RPW // 16)
        def _(i):
            e16 = e_v[pl.ds(i * 16, 16)]
            t = i // (K // 16)
            lane = (i % (K // 16)) * 16
            idx16 = idx_v[t, pl.ds(lane, 16)]
            hi = lax.shift_right_logical(idx16, 7)
            lo = lax.bitwise_and(idx16, 127)
            plsc.addupdate_scatter(den_l, [hi, lo], e16)

        rows16 = lax.iota(jnp.int32, 16)
        pltpu.sync_copy(den_l, den_s.at[rows16], add=True)
        plsc.subcore_barrier()

        pltpu.sync_copy(
            acc.at[pl.ds(sid * 64, 64)],
            num_out.at[cid, pl.ds(sid * 64, 64)],
        )

        @pl.when(sid == 0)
        def _():
            pltpu.sync_copy(
                acc.at[pl.ds(1024, GA - 1024)],
                num_out.at[cid, pl.ds(1024, GA - 1024)],
            )

        pltpu.sync_copy(den_s, den_m)

        lanes16 = lax.iota(jnp.int32, 16)

        def bcast_16rows(g0, buf_row0):
            v = den_m[g0 // 128, pl.ds((g0 % 128) // 16 * 16, 16)]
            for r in range(16):
                oh = (lanes16 == r).astype(jnp.float32)
                row = jnp.full((16,), jnp.sum(v * oh), jnp.float32)
                for c in range(D // 16):
                    buf[0, buf_row0 + r, pl.ds(c * 16, 16)] = row

        sbase = sid * 64

        @pl.loop(0, 4)
        def _(q):
            bcast_16rows(sbase + q * 16, q * 16)

        pltpu.sync_copy(
            buf.at[0, pl.ds(0, 64)],
            den_out.at[cid, pl.ds(sid * 64, 64)],
        )

        @pl.when(sid == 0)
        def _():
            bcast_16rows(1024, 0)
            pltpu.sync_copy(
                buf.at[0, pl.ds(0, GA - 1024)],
                den_out.at[cid, pl.ds(1024, GA - 1024)],
            )

    return scatter


def kernel(x, batch_idx, W1, b1, W2, b2):
    idx3 = jnp.pad(batch_idx.astype(jnp.int32), (0, N_PAD - N),
                   constant_values=G).reshape(NW, T, K)
    w2t = W2.reshape(1, H)
    b1r = b1.reshape(1, H)
    b2r = b2.reshape(1, 1)

    ex, e_rows = pl.pallas_call(
        _score_kernel,
        grid=(NW,),
        in_specs=[
            pl.BlockSpec((RPW, D), lambda i: (i, 0)),
            pl.BlockSpec((D, H), lambda i: (0, 0)),
            pl.BlockSpec((1, H), lambda i: (0, 0)),
            pl.BlockSpec((1, H), lambda i: (0, 0)),
            pl.BlockSpec((1, 1), lambda i: (0, 0)),
        ],
        out_specs=[
            pl.BlockSpec((RPW, D), lambda i: (i, 0)),
            pl.BlockSpec((1, 1, RPW), lambda i: (i, 0, 0)),
        ],
        out_shape=[
            jax.ShapeDtypeStruct((N_PAD, D), jnp.float32),
            jax.ShapeDtypeStruct((NW, 1, RPW), jnp.float32),
        ],
        compiler_params=pltpu.CompilerParams(
            dimension_semantics=("parallel",)),
    )(x, W1, b1r, w2t, b2r)
    e_rows = e_rows.reshape(NW, RPW)

    zeros = jnp.zeros((GA, D), jnp.float32)
    num_p, den_p = _make_scatter_kernel()(ex, idx3, e_rows, zeros)

    pooled = pl.pallas_call(
        _final_kernel,
        grid=(1,),
        in_specs=[
            pl.BlockSpec((2, GA, D), lambda i: (0, 0, 0)),
            pl.BlockSpec((2, GA, D), lambda i: (0, 0, 0)),
        ],
        out_specs=pl.BlockSpec((G, D), lambda i: (0, 0)),
        out_shape=jax.ShapeDtypeStruct((G, D), jnp.float32),
    )(num_p, den_p)
    return pooled


# speedup vs baseline: 12.7499x; 1.0010x over previous
import dataclasses
import functools

import jax
import jax.numpy as jnp
from jax import lax
from jax.experimental import pallas as pl
from jax.experimental.pallas import tpu as pltpu
from jax.experimental.pallas import tpu_sc as plsc

N = 100000
D = 128
H = 64
G = 1024

NW = 32
K = 80
TPW = 40
RPW = TPW * K
NCH = NW * RPW
C = 1
N_PAD = C * NCH
NBUF = 8
GA = 1040
DEN_R = 16


def _score_kernel(x_ref, w1_ref, b1_ref, w2_ref, b2_ref, ex_ref, e_ref):
    x = x_ref[...]
    w2 = w2_ref[...]
    b2 = b2_ref[0, 0]
    m = jnp.sum(jnp.abs(w2)) + jnp.abs(b2)
    h = jnp.tanh(
        jnp.dot(x, w1_ref[...], preferred_element_type=jnp.float32)
        + b1_ref[...]
    )
    s_col = jnp.sum(h * w2, axis=1, keepdims=True) + b2
    ex_ref[...] = x * jnp.exp(s_col - m)
    s_row = lax.dot_general(
        w2, h, (((1,), (1,)), ((), ())),
        preferred_element_type=jnp.float32) + b2
    e_ref[...] = jnp.exp(s_row - m).reshape(1, 1, RPW)


def _final_kernel(*refs):
    out_ref = refs[-1]
    p_refs = refs[:C]
    d_refs = refs[C:2 * C]
    num = sum(p[0] + p[1] for p in p_refs)[0:G, :]
    den = sum(d[0] + d[1] for d in d_refs)[0:G, :]
    out_ref[...] = jnp.where(den > 0.0, num / den, 0.0)


@functools.lru_cache(maxsize=None)
def _make_scatter_kernel():
    mesh = plsc.VectorSubcoreMesh(core_axis_name="c", subcore_axis_name="s")
    cp = pltpu.CompilerParams()
    if "needs_layout_passes" in pltpu.CompilerParams.__dataclass_fields__:
        cp = dataclasses.replace(cp, needs_layout_passes=False)

    @functools.partial(
        pl.kernel,
        compiler_params=cp,
        out_type=(
            jax.ShapeDtypeStruct((2, GA, D), jnp.float32),
            jax.ShapeDtypeStruct((2, GA, D), jnp.float32),
        ),
        mesh=mesh,
        scratch_types=[
            pltpu.VMEM((NBUF, K, D), jnp.float32),
            pltpu.VMEM((TPW, K), jnp.int32),
            pltpu.VMEM((RPW,), jnp.float32),
            pltpu.VMEM((DEN_R, D), jnp.float32),
            pltpu.VMEM((DEN_R, D), jnp.float32),
            pltpu.VMEM_SHARED((GA, D), jnp.float32),
            pltpu.VMEM_SHARED((DEN_R, D), jnp.float32),
            pltpu.SemaphoreType.DMA,
            pltpu.SemaphoreType.DMA((4,)),
            pltpu.SemaphoreType.DMA((4,)),
        ],
    )
    def scatter(ex_hbm, idx_hbm, e_hbm, zeros_hbm, num_out, den_out,
                buf, idx_v, e_v, den_l, den_m, acc, den_s, sem, lsem, ssem):
        cid = lax.axis_index("c")
        sid = lax.axis_index("s")
        wid = cid * 16 + sid

        @pl.when(sid == 0)
        def _():
            pltpu.sync_copy(zeros_hbm, acc)
            pltpu.sync_copy(zeros_hbm.at[pl.ds(0, DEN_R)], den_s)

        pltpu.sync_copy(idx_hbm.at[wid], idx_v)
        pltpu.sync_copy(e_hbm.at[wid], e_v)

        zero16 = jnp.zeros((16,), jnp.float32)
        for r in range(DEN_R):
            for c in range(D // 16):
                den_l[r, pl.ds(c * 16, 16)] = zero16

        plsc.subcore_barrier()

        base = wid * RPW

        def load_copy(t):
            return pltpu.make_async_copy(
                ex_hbm.at[pl.ds(base + t * K, K)], buf.at[t % NBUF],
                lsem.at[t % 4])

        def scat_copy(t):
            return pltpu.make_async_copy(
                buf.at[t % NBUF], acc.at[idx_v.at[t]], ssem.at[t % 4])

        for t in range(4):
            load_copy(t).start()

        @pl.loop(0, TPW // 2)
        def _(p):
            for t in (2 * p, 2 * p + 1):
                load_copy(t).wait()

                @pl.when(p >= 2)
                def _():
                    scat_copy(t - 4).wait()

                scat_copy(t).start(add=True)

                @pl.when(p < TPW // 2 - 2)
                def _():
                    load_copy(t + 4).start()

        for q in range(4):
            scat_copy(TPW - 4 + q).wait()

        @pl.loop(0, RPW // 16)
        def _(i):
            e16 = e_v[pl.ds(i * 16, 16)]
            t = i // (K // 16)
            lane = (i % (K // 16)) * 16
            idx16 = idx_v[t, pl.ds(lane, 16)]
            hi = lax.shift_right_logical(idx16, 7)
            lo = lax.bitwise_and(idx16, 127)
            plsc.addupdate_scatter(den_l, [hi, lo], e16)

        rows16 = lax.iota(jnp.int32, 16)
        pltpu.sync_copy(den_l, den_s.at[rows16], add=True)
        plsc.subcore_barrier()

        pltpu.sync_copy(
            acc.at[pl.ds(sid * 64, 64)],
            num_out.at[cid, pl.ds(sid * 64, 64)],
        )

        @pl.when(sid == 0)
        def _():
            pltpu.sync_copy(
                acc.at[pl.ds(1024, GA - 1024)],
                num_out.at[cid, pl.ds(1024, GA - 1024)],
            )

        pltpu.sync_copy(den_s, den_m)

        lanes16 = lax.iota(jnp.int32, 16)

        def bcast_16rows(g0, buf_row0):
            v = den_m[g0 // 128, pl.ds((g0 % 128) // 16 * 16, 16)]
            for r in range(16):
                oh = (lanes16 == r).astype(jnp.float32)
                row = jnp.full((16,), jnp.sum(v * oh), jnp.float32)
                for cc in range(D // 16):
                    buf[0, buf_row0 + r, pl.ds(cc * 16, 16)] = row

        sbase = sid * 64

        @pl.loop(0, 4)
        def _(q):
            bcast_16rows(sbase + q * 16, q * 16)

        pltpu.sync_copy(
            buf.at[0, pl.ds(0, 64)],
            den_out.at[cid, pl.ds(sid * 64, 64)],
        )

        @pl.when(sid == 0)
        def _():
            bcast_16rows(1024, 0)
            pltpu.sync_copy(
                buf.at[0, pl.ds(0, GA - 1024)],
                den_out.at[cid, pl.ds(1024, GA - 1024)],
            )

    return scatter


def kernel(x, batch_idx, W1, b1, W2, b2):
    idx4 = jnp.pad(batch_idx.astype(jnp.int32), (0, N_PAD - N),
                   constant_values=G).reshape(C, NW, TPW, K)
    w2t = W2.reshape(1, H)
    b1r = b1.reshape(1, H)
    b2r = b2.reshape(1, 1)
    zeros = jnp.zeros((GA, D), jnp.float32)

    num_p = []
    den_p = []
    zeros_c = zeros
    for c in range(C):
        ex, e_rows = pl.pallas_call(
            _score_kernel,
            grid=(NW,),
            in_specs=[
                pl.BlockSpec((RPW, D), lambda i, c=c: (i + c * NW, 0)),
                pl.BlockSpec((D, H), lambda i: (0, 0)),
                pl.BlockSpec((1, H), lambda i: (0, 0)),
                pl.BlockSpec((1, H), lambda i: (0, 0)),
                pl.BlockSpec((1, 1), lambda i: (0, 0)),
            ],
            out_specs=[
                pl.BlockSpec((RPW, D), lambda i: (i, 0)),
                pl.BlockSpec((1, 1, RPW), lambda i: (i, 0, 0)),
            ],
            out_shape=[
                jax.ShapeDtypeStruct((NCH, D), jnp.float32),
                jax.ShapeDtypeStruct((NW, 1, RPW), jnp.float32),
            ],
        )(x, W1, b1r, w2t, b2r)
        e_rows = e_rows.reshape(NW, RPW)
        np_c, dp_c = _make_scatter_kernel()(ex, idx4[c], e_rows, zeros_c)
        num_p.append(np_c)
        den_p.append(dp_c)
        zeros_c, _ = lax.optimization_barrier((zeros, np_c))

    pooled = pl.pallas_call(
        _final_kernel,
        grid=(1,),
        in_specs=[pl.BlockSpec((2, GA, D), lambda i: (0, 0, 0))] * (2 * C),
        out_specs=pl.BlockSpec((G, D), lambda i: (0, 0)),
        out_shape=jax.ShapeDtypeStruct((G, D), jnp.float32),
    )(*num_p, *den_p)
    return pooled
